# baseline (speedup 1.0000x reference)
import functools

import jax
import jax.numpy as jnp
from jax import lax
from jax.experimental import pallas as pl
from jax.experimental.pallas import tpu as pltpu
from jax.experimental.pallas import tpu_sc as plsc

N = 10000
E = 320000
G = 64
NC = 2
NS = 16
C = 40
SB = 10
NBUF = 5
GA = 3
NP = 10240
RPS = NP // NS
BR = 1000
NB = N // BR
CH = 256
CH2 = 400
NEG_INF = float("-inf")


def _make_spmm(edge_split):
  D2 = 128
  nf = D2 // 16
  K = (E // NC if edge_split else E) // NS // C
  NSB = K // SB
  mesh = plsc.VectorSubcoreMesh(core_axis_name="c", subcore_axis_name="s")

  scratch = (
      [pltpu.VMEM((3, 2, SB, C), jnp.int32)]
      + [pltpu.VMEM((2 * C, D2), jnp.float32)]
      + [pltpu.VMEM((2 * C, D2), jnp.float32)]
      + [pltpu.VMEM((2 * C, 16), jnp.float32)]
      + [pltpu.VMEM_SHARED((NP, D2), jnp.float32)]
      + [pltpu.SemaphoreType.DMA for _ in range(6)]
  )

  @functools.partial(pl.kernel,
                     out_type=jax.ShapeDtypeStruct((NC, NP, D2), jnp.float32),
                     mesh=mesh, scratch_types=scratch,
                     compiler_params=pltpu.CompilerParams(
                         needs_layout_passes=False))
  def spmm(h_hbm, pk_hbm, ew16_hbm, out_hbm, pbuf, gball, sball, ewall,
           slab, g0, g1, s0, s1, e0, e1):
    gbuf = [gball.at[pl.ds(b * C, C)] for b in range(2)]
    sbuf = [sball.at[pl.ds(b * C, C)] for b in range(2)]
    ewb = [ewall.at[pl.ds(b * C, C)] for b in range(2)]
    gsem = (g0, g1)
    ssem = (s0, s1)
    esem = (e0, e1)

    cidx = lax.axis_index("c")
    sid = lax.axis_index("s")
    h_c = h_hbm if edge_split else h_hbm.at[cidx]
    out_c = out_hbm.at[cidx]
    pk_s = pk_hbm.at[cidx, sid] if edge_split else pk_hbm.at[sid]
    ew_s = ew16_hbm.at[cidx, sid] if edge_split else ew16_hbm.at[sid]

    def load_window(w):
      pltpu.sync_copy(pk_s.at[w], pbuf.at[lax.rem(w, 3)])

    @pl.loop(0, C)
    def _(e):
      for f in range(nf):
        sball[e, pl.ds(f * 16, 16)] = jnp.zeros((16,), jnp.float32)

    base = sid * RPS

    @pl.loop(0, RPS // C)
    def _(i):
      pltpu.sync_copy(sbuf[0], slab.at[pl.ds(base + i * C, C)])

    load_window(0)
    load_window(1)

    plsc.subcore_barrier()

    def chunk_idx(c):
      w = lax.div(c, SB)
      return lax.rem(w, 3), lax.rem(c, SB)

    def issue_in(b, c):
      bank, ck = chunk_idx(c)
      pltpu.async_copy(h_c.at[pbuf.at[bank, 0, ck]], gbuf[b], gsem[b])
      pltpu.async_copy(ew_s.at[c], ewb[b], esem[b])

    def wait_in(b, c):
      bank, ck = chunk_idx(c)
      pltpu.make_async_copy(h_c.at[pbuf.at[bank, 0, ck]], gbuf[b],
                            gsem[b]).wait()
      pltpu.make_async_copy(ew_s.at[c], ewb[b], esem[b]).wait()

    def issue_out(b, c):
      bank, ck = chunk_idx(c)
      pltpu.async_copy(sbuf[b], slab.at[pbuf.at[bank, 1, ck]], ssem[b],
                       add=True)

    def wait_out(b, c):
      bank, ck = chunk_idx(c)
      pltpu.make_async_copy(sbuf[b], slab.at[pbuf.at[bank, 1, ck]],
                            ssem[b]).wait()

    for b in range(2):
      issue_in(b, b)

    @pl.loop(0, K, step=2)
    def _(j0):
      for b in range(2):
        c = j0 + b
        if b == 0:
          @pl.when(lax.rem(c, SB) == 0)
          def _():
            @pl.when(lax.div(c, SB) + 1 < NSB)
            def _():
              load_window(lax.div(c, SB) + 1)

        wait_in(b, c)

        @pl.when(c >= 2)
        def _():
          wait_out(b, c - 2)

        @pl.loop(0, C, unroll=8)
        def _(e):
          wv = ewall[b * C + e, :]
          for f in range(nf):
            sl = pl.ds(f * 16, 16)
            sball[b * C + e, sl] = gball[b * C + e, sl] * wv

        @pl.when(c + 2 < K)
        def _():
          issue_in(b, c + 2)

        issue_out(b, c)

    for b in range(2):
      wait_out(b, K - 2 + b)

    plsc.subcore_barrier()

    pltpu.sync_copy(slab.at[pl.ds(base, RPS)], out_c.at[pl.ds(base, RPS)])

  return spmm


def _dense_body(agg_split, hp_split, Din, Dout, agg_ref, hp_ref, Wr_ref,
                Ws_ref, b_ref, t_ref, ssum_ref, ssq_ref):
  Dh = Din // 2
  j = pl.program_id(0)

  Wr = Wr_ref[...]
  Ws = Ws_ref[...]

  if agg_split:
    ts = (jnp.dot(agg_ref[0], Wr[:Dh, :], preferred_element_type=jnp.float32)
          + jnp.dot(agg_ref[1], Wr[Dh:, :], preferred_element_type=jnp.float32))
  else:
    ts = jnp.dot(agg_ref[0] + agg_ref[1], Wr,
                 preferred_element_type=jnp.float32)

  if hp_split:
    ts = (ts
          + jnp.dot(hp_ref[0], Ws[:Dh, :], preferred_element_type=jnp.float32)
          + jnp.dot(hp_ref[1], Ws[Dh:, :], preferred_element_type=jnp.float32))
  else:
    ts = ts + jnp.dot(hp_ref[...], Ws, preferred_element_type=jnp.float32)

  ts = ts + b_ref[...]
  t = jnp.where(ts > 0, ts, 0.01 * ts)
  t_ref[...] = t

  bs = jnp.sum(t, axis=0, keepdims=True)
  bq = jnp.sum(t * t, axis=0, keepdims=True)

  @pl.when(j == 0)
  def _():
    ssum_ref[...] = bs
    ssq_ref[...] = bq

  @pl.when(j > 0)
  def _():
    ssum_ref[...] += bs
    ssq_ref[...] += bq


def _make_dense(agg_split, hp_split, Din, Dout):
  Dh = Din // 2
  body = functools.partial(_dense_body, agg_split, hp_split, Din, Dout)
  if agg_split:
    agg_spec = pl.BlockSpec((2, BR, Dh), lambda j: (0, j, 0))
  else:
    agg_spec = pl.BlockSpec((2, BR, Din), lambda j: (0, j, 0))
  if hp_split:
    hp_spec = pl.BlockSpec((2, BR, Dh), lambda j: (0, j, 0))
  else:
    hp_spec = pl.BlockSpec((BR, Din), lambda j: (j, 0))
  return pl.pallas_call(
      body,
      grid=(NB,),
      in_specs=[
          agg_spec,
          hp_spec,
          pl.BlockSpec((Din, Dout), lambda j: (0, 0)),
          pl.BlockSpec((Din, Dout), lambda j: (0, 0)),
          pl.BlockSpec((1, Dout), lambda j: (0, 0)),
      ],
      out_specs=[
          pl.BlockSpec((BR, Dout), lambda j: (j, 0)),
          pl.BlockSpec((1, Dout), lambda j: (0, 0)),
          pl.BlockSpec((1, Dout), lambda j: (0, 0)),
      ],
      out_shape=[
          jax.ShapeDtypeStruct((N, Dout), jnp.float32),
          jax.ShapeDtypeStruct((1, Dout), jnp.float32),
          jax.ShapeDtypeStruct((1, Dout), jnp.float32),
      ],
  )


def _norm_body(hn_mode, Dout, t_ref, ssum_ref, ssq_ref, g_ref, be_ref,
               hn_ref):
  Do2 = Dout // 2
  m = ssum_ref[...] / N
  var = ssq_ref[...] / N - m * m
  istd = lax.rsqrt(var + 1e-5)
  a = g_ref[...] * istd
  cc = be_ref[...] - m * a
  hn = a * t_ref[...] + cc
  if hn_mode == "full":
    hn_ref[...] = hn
  else:
    hn_ref[0] = hn[:, :Do2]
    hn_ref[1] = hn[:, Do2:]


def _make_norm(hn_mode, Dout):
  Do2 = Dout // 2
  body = functools.partial(_norm_body, hn_mode, Dout)
  NBN = N // CH2
  if hn_mode == "full":
    hn_spec = pl.BlockSpec((CH2, Dout), lambda j: (j, 0))
    hn_shape = jax.ShapeDtypeStruct((N, Dout), jnp.float32)
  else:
    hn_spec = pl.BlockSpec((2, CH2, Do2), lambda j: (0, j, 0))
    hn_shape = jax.ShapeDtypeStruct((2, N, Do2), jnp.float32)
  return pl.pallas_call(
      body,
      grid=(NBN,),
      in_specs=[
          pl.BlockSpec((CH2, Dout), lambda j: (j, 0)),
          pl.BlockSpec((1, Dout), lambda j: (0, 0)),
          pl.BlockSpec((1, Dout), lambda j: (0, 0)),
          pl.BlockSpec((1, Dout), lambda j: (0, 0)),
          pl.BlockSpec((1, Dout), lambda j: (0, 0)),
      ],
      out_specs=hn_spec,
      out_shape=hn_shape,
  )


def _pool_body(Dout, t_ref, ssum_ref, ssq_ref, g_ref, be_ref,
               st_ref, ct_ref, pool_ref):
  m = ssum_ref[...] / N
  var = ssq_ref[...] / N - m * m
  istd = lax.rsqrt(var + 1e-5)
  a = g_ref[...] * istd
  cc = be_ref[...] - m * a

  CW = CH + 8

  for g in range(G):
    st = st_ref[g]
    cnt = ct_ref[g]
    nchunk = (cnt + CH - 1) // CH

    def chunk_body(i, carry, st=st, cnt=cnt):
      mx, mn, sm = carry
      off = st + i * CH
      offc = jnp.minimum(off, N - CW)
      offc = pl.multiple_of((offc // 8) * 8, 8)
      rows = offc + lax.broadcasted_iota(jnp.int32, (CW, 1), 0)
      mask = (rows >= off) & (rows < off + CH) & (rows < st + cnt)
      vals = t_ref[pl.ds(offc, CW), :]
      mx = jnp.maximum(mx, jnp.max(jnp.where(mask, vals, NEG_INF), axis=0,
                                   keepdims=True))
      mn = jnp.minimum(mn, jnp.min(jnp.where(mask, vals, jnp.inf), axis=0,
                                   keepdims=True))
      sm = sm + jnp.sum(jnp.where(mask, vals, 0.0), axis=0, keepdims=True)
      return mx, mn, sm

    init = (jnp.full((1, Dout), NEG_INF, jnp.float32),
            jnp.full((1, Dout), jnp.inf, jnp.float32),
            jnp.zeros((1, Dout), jnp.float32))
    mx, mn, sm = lax.fori_loop(0, nchunk, chunk_body, init)

    nonempty = cnt > 0
    hmx = jnp.where(a >= 0, a * mx + cc, a * mn + cc)
    hmx = jnp.where(nonempty, hmx, 0.0)
    hsm = a * sm + cc * cnt.astype(jnp.float32)
    hmean = hsm / jnp.maximum(cnt.astype(jnp.float32), 1.0)
    out = jnp.concatenate([hmx, hmean, hsm], axis=1)
    pool_ref[g:g + 1, :] = out


def _make_pool(Dout):
  body = functools.partial(_pool_body, Dout)
  return pl.pallas_call(
      body,
      in_specs=[
          pl.BlockSpec(memory_space=pltpu.VMEM),
          pl.BlockSpec(memory_space=pltpu.VMEM),
          pl.BlockSpec(memory_space=pltpu.VMEM),
          pl.BlockSpec(memory_space=pltpu.VMEM),
          pl.BlockSpec(memory_space=pltpu.VMEM),
          pl.BlockSpec(memory_space=pltpu.SMEM),
          pl.BlockSpec(memory_space=pltpu.SMEM),
      ],
      out_specs=pl.BlockSpec(memory_space=pltpu.VMEM),
      out_shape=jax.ShapeDtypeStruct((G, 3 * Dout), jnp.float32),
  )


def _head_body(x1_ref, x2_ref, x3_ref, Wl1_ref, bl1_ref, Wl2_ref, bl2_ref,
               Wl3_ref, bl3_ref, out_ref):
  z = jnp.concatenate([x1_ref[...], x2_ref[...], x3_ref[...]], axis=1)
  z = jnp.dot(z, Wl1_ref[...], preferred_element_type=jnp.float32) + bl1_ref[...]
  z = jnp.maximum(z, 0.0)
  z = jnp.dot(z, Wl2_ref[...], preferred_element_type=jnp.float32) + bl2_ref[...]
  z = jnp.maximum(z, 0.0)
  z = jnp.dot(z, Wl3_ref[...], preferred_element_type=jnp.float32) + bl3_ref[...]
  out_ref[...] = jax.nn.log_softmax(z, axis=-1)


_spmm12 = _make_spmm(True)
_spmm3 = _make_spmm(False)
_dense1 = _make_dense(False, False, 128, 128)
_dense2 = _make_dense(False, False, 128, 256)
_dense3 = _make_dense(True, True, 256, 384)
_norm1 = _make_norm("full", 128)
_norm2 = _make_norm("split", 256)
_pool1 = _make_pool(128)
_pool2 = _make_pool(256)
_pool3 = _make_pool(384)

KES = (E // NC) // NS // C
KFS = E // NS // C


def kernel(x, edge_index, batch, edge_attr, W1r, W1s, b1, W2r, W2s, b2,
           W3r, W3s, b3, g1, be1, g2, be2, g3, be3, Wl1, bl1, Wl2, bl2,
           Wl3, bl3):
  ew = edge_attr[:, 0]
  pk_e = jnp.transpose(
      edge_index.reshape(2, NC, NS, KES // SB, SB, C), (1, 2, 3, 0, 4, 5))
  pk_f = jnp.transpose(
      edge_index.reshape(2, NS, KFS // SB, SB, C), (1, 2, 0, 3, 4))
  ew16 = jnp.broadcast_to(ew[:, None], (E, 16))
  ew16_e = ew16.reshape(NC, NS, KES, C, 16)
  ew16_f = ew16.reshape(NS, KFS, C, 16)

  starts = jnp.searchsorted(batch, jnp.arange(G, dtype=batch.dtype)
                            ).astype(jnp.int32)
  ends = jnp.searchsorted(batch, jnp.arange(G, dtype=batch.dtype),
                          side="right").astype(jnp.int32)
  counts = ends - starts

  agg1 = _spmm12(x, pk_e, ew16_e)
  t1, s1, q1 = _dense1(agg1, x, W1r, W1s, b1[None, :])
  hn1 = _norm1(t1, s1, q1, g1[None, :], be1[None, :])
  x1p = _pool1(t1, s1, q1, g1[None, :], be1[None, :], starts, counts)

  agg2 = _spmm12(hn1, pk_e, ew16_e)
  t2, s2, q2 = _dense2(agg2, hn1, W2r, W2s, b2[None, :])
  hn2 = _norm2(t2, s2, q2, g2[None, :], be2[None, :])
  x2p = _pool2(t2, s2, q2, g2[None, :], be2[None, :], starts, counts)

  agg3 = _spmm3(hn2, pk_f, ew16_f)
  t3, s3, q3 = _dense3(agg3, hn2, W3r, W3s, b3[None, :])
  x3p = _pool3(t3, s3, q3, g3[None, :], be3[None, :], starts, counts)

  out = pl.pallas_call(
      _head_body,
      out_shape=jax.ShapeDtypeStruct((G, 2), jnp.float32),
  )(x1p, x2p, x3p, Wl1, bl1[None, :], Wl2, bl2[None, :], Wl3, bl3[None, :])
  return out


# speedup vs baseline: 1.8013x; 1.8013x over previous
import functools

import jax
import jax.numpy as jnp
from jax import lax
from jax.experimental import pallas as pl
from jax.experimental.pallas import tpu as pltpu
from jax.experimental.pallas import tpu_sc as plsc

N = 10000
E = 320000
G = 64
NC = 2
NS = 16
C = 40
SB = 10
NBUF = 5
GA = 3
NP = 10240
RPS = NP // NS
BR = 1000
NB = N // BR
CH = 256
CH2 = 400
NEG_INF = float("-inf")


def _make_spmm(edge_split):
  D2 = 128
  nf = D2 // 16
  K = (E // NC if edge_split else E) // NS // C
  NSB = K // SB
  mesh = plsc.VectorSubcoreMesh(core_axis_name="c", subcore_axis_name="s")

  scratch = (
      [pltpu.VMEM((3, 2, SB, C), jnp.int32)]
      + [pltpu.VMEM((2 * C, D2), jnp.float32)]
      + [pltpu.VMEM((2 * C, D2), jnp.float32)]
      + [pltpu.VMEM((2 * C, 16), jnp.float32)]
      + [pltpu.VMEM_SHARED((NP, D2), jnp.float32)]
      + [pltpu.SemaphoreType.DMA for _ in range(6)]
  )

  @functools.partial(pl.kernel,
                     out_type=jax.ShapeDtypeStruct((NC, NP, D2), jnp.float32),
                     mesh=mesh, scratch_types=scratch,
                     compiler_params=pltpu.CompilerParams(
                         needs_layout_passes=False))
  def spmm(h_hbm, pk_hbm, ew16_hbm, out_hbm, pbuf, gball, sball, ewall,
           slab, g0, g1, s0, s1, e0, e1):
    gbuf = [gball.at[pl.ds(b * C, C)] for b in range(2)]
    sbuf = [sball.at[pl.ds(b * C, C)] for b in range(2)]
    ewb = [ewall.at[pl.ds(b * C, C)] for b in range(2)]
    gsem = (g0, g1)
    ssem = (s0, s1)
    esem = (e0, e1)

    cidx = lax.axis_index("c")
    sid = lax.axis_index("s")
    h_c = h_hbm if edge_split else h_hbm.at[cidx]
    out_c = out_hbm.at[cidx]
    pk_s = pk_hbm.at[cidx, sid] if edge_split else pk_hbm.at[sid]
    ew_s = ew16_hbm.at[cidx, sid] if edge_split else ew16_hbm.at[sid]

    def load_window(w):
      pltpu.sync_copy(pk_s.at[w], pbuf.at[lax.rem(w, 3)])

    @pl.loop(0, C)
    def _(e):
      for f in range(nf):
        sball[e, pl.ds(f * 16, 16)] = jnp.zeros((16,), jnp.float32)

    base = sid * RPS

    @pl.loop(0, RPS // C)
    def _(i):
      pltpu.sync_copy(sbuf[0], slab.at[pl.ds(base + i * C, C)])

    load_window(0)
    load_window(1)

    plsc.subcore_barrier()

    def chunk_idx(c):
      w = lax.div(c, SB)
      return lax.rem(w, 3), lax.rem(c, SB)

    def issue_in(b, c):
      bank, ck = chunk_idx(c)
      pltpu.async_copy(h_c.at[pbuf.at[bank, 0, ck]], gbuf[b], gsem[b])
      pltpu.async_copy(ew_s.at[c], ewb[b], esem[b])

    def wait_in(b, c):
      bank, ck = chunk_idx(c)
      pltpu.make_async_copy(h_c.at[pbuf.at[bank, 0, ck]], gbuf[b],
                            gsem[b]).wait()
      pltpu.make_async_copy(ew_s.at[c], ewb[b], esem[b]).wait()

    def issue_out(b, c):
      bank, ck = chunk_idx(c)
      pltpu.async_copy(sbuf[b], slab.at[pbuf.at[bank, 1, ck]], ssem[b],
                       add=True)

    def wait_out(b, c):
      bank, ck = chunk_idx(c)
      pltpu.make_async_copy(sbuf[b], slab.at[pbuf.at[bank, 1, ck]],
                            ssem[b]).wait()

    for b in range(2):
      issue_in(b, b)

    @pl.loop(0, K, step=2)
    def _(j0):
      for b in range(2):
        c = j0 + b
        if b == 0:
          @pl.when(lax.rem(c, SB) == 0)
          def _():
            @pl.when(lax.div(c, SB) + 1 < NSB)
            def _():
              load_window(lax.div(c, SB) + 1)

        wait_in(b, c)

        @pl.when(c >= 2)
        def _():
          wait_out(b, c - 2)

        @pl.loop(0, C, unroll=C)
        def _(e):
          wv = ewall[b * C + e, :]
          for f in range(nf):
            sl = pl.ds(f * 16, 16)
            sball[b * C + e, sl] = gball[b * C + e, sl] * wv

        @pl.when(c + 2 < K)
        def _():
          issue_in(b, c + 2)

        issue_out(b, c)

    for b in range(2):
      wait_out(b, K - 2 + b)

    plsc.subcore_barrier()

    pltpu.sync_copy(slab.at[pl.ds(base, RPS)], out_c.at[pl.ds(base, RPS)])

  return spmm


def _dense_body(agg_split, hp_split, Din, Dout, agg_ref, hp_ref, Wr_ref,
                Ws_ref, b_ref, t_ref, ssum_ref, ssq_ref):
  Dh = Din // 2
  j = pl.program_id(0)

  Wr = Wr_ref[...]
  Ws = Ws_ref[...]

  if agg_split:
    ts = (jnp.dot(agg_ref[0], Wr[:Dh, :], preferred_element_type=jnp.float32)
          + jnp.dot(agg_ref[1], Wr[Dh:, :], preferred_element_type=jnp.float32))
  else:
    ts = jnp.dot(agg_ref[0] + agg_ref[1], Wr,
                 preferred_element_type=jnp.float32)

  if hp_split:
    ts = (ts
          + jnp.dot(hp_ref[0], Ws[:Dh, :], preferred_element_type=jnp.float32)
          + jnp.dot(hp_ref[1], Ws[Dh:, :], preferred_element_type=jnp.float32))
  else:
    ts = ts + jnp.dot(hp_ref[...], Ws, preferred_element_type=jnp.float32)

  ts = ts + b_ref[...]
  t = jnp.where(ts > 0, ts, 0.01 * ts)
  t_ref[...] = t

  bs = jnp.sum(t, axis=0, keepdims=True)
  bq = jnp.sum(t * t, axis=0, keepdims=True)

  @pl.when(j == 0)
  def _():
    ssum_ref[...] = bs
    ssq_ref[...] = bq

  @pl.when(j > 0)
  def _():
    ssum_ref[...] += bs
    ssq_ref[...] += bq


def _make_dense(agg_split, hp_split, Din, Dout):
  Dh = Din // 2
  body = functools.partial(_dense_body, agg_split, hp_split, Din, Dout)
  if agg_split:
    agg_spec = pl.BlockSpec((2, BR, Dh), lambda j: (0, j, 0))
  else:
    agg_spec = pl.BlockSpec((2, BR, Din), lambda j: (0, j, 0))
  if hp_split:
    hp_spec = pl.BlockSpec((2, BR, Dh), lambda j: (0, j, 0))
  else:
    hp_spec = pl.BlockSpec((BR, Din), lambda j: (j, 0))
  return pl.pallas_call(
      body,
      grid=(NB,),
      in_specs=[
          agg_spec,
          hp_spec,
          pl.BlockSpec((Din, Dout), lambda j: (0, 0)),
          pl.BlockSpec((Din, Dout), lambda j: (0, 0)),
          pl.BlockSpec((1, Dout), lambda j: (0, 0)),
      ],
      out_specs=[
          pl.BlockSpec((BR, Dout), lambda j: (j, 0)),
          pl.BlockSpec((1, Dout), lambda j: (0, 0)),
          pl.BlockSpec((1, Dout), lambda j: (0, 0)),
      ],
      out_shape=[
          jax.ShapeDtypeStruct((N, Dout), jnp.float32),
          jax.ShapeDtypeStruct((1, Dout), jnp.float32),
          jax.ShapeDtypeStruct((1, Dout), jnp.float32),
      ],
  )


def _pool_body(hn_mode, Dout, t_ref, ssum_ref, ssq_ref, g_ref, be_ref,
               st_ref, ct_ref, pool_ref, hn_ref):
  Do2 = Dout // 2
  m = ssum_ref[...] / N
  var = ssq_ref[...] / N - m * m
  istd = lax.rsqrt(var + 1e-5)
  a = g_ref[...] * istd
  cc = be_ref[...] - m * a

  if hn_mode != "none":
    def norm_body(i, _):
      sl = pl.ds(i * CH2, CH2)
      hn = a * t_ref[sl, :] + cc
      if hn_mode == "full":
        hn_ref[sl, :] = hn
      else:
        hn_ref[0, sl, :] = hn[:, :Do2]
        hn_ref[1, sl, :] = hn[:, Do2:]
      return 0
    lax.fori_loop(0, N // CH2, norm_body, 0)

  CW = CH + 8

  for g in range(G):
    st = st_ref[g]
    cnt = ct_ref[g]
    nchunk = (cnt + CH - 1) // CH

    def chunk_body(i, carry, st=st, cnt=cnt):
      mx, mn, sm = carry
      off = st + i * CH
      offc = jnp.minimum(off, N - CW)
      offc = pl.multiple_of((offc // 8) * 8, 8)
      rows = offc + lax.broadcasted_iota(jnp.int32, (CW, 1), 0)
      mask = (rows >= off) & (rows < off + CH) & (rows < st + cnt)
      vals = t_ref[pl.ds(offc, CW), :]
      mx = jnp.maximum(mx, jnp.max(jnp.where(mask, vals, NEG_INF), axis=0,
                                   keepdims=True))
      mn = jnp.minimum(mn, jnp.min(jnp.where(mask, vals, jnp.inf), axis=0,
                                   keepdims=True))
      sm = sm + jnp.sum(jnp.where(mask, vals, 0.0), axis=0, keepdims=True)
      return mx, mn, sm

    init = (jnp.full((1, Dout), NEG_INF, jnp.float32),
            jnp.full((1, Dout), jnp.inf, jnp.float32),
            jnp.zeros((1, Dout), jnp.float32))
    mx, mn, sm = lax.fori_loop(0, nchunk, chunk_body, init)

    nonempty = cnt > 0
    hmx = jnp.where(a >= 0, a * mx + cc, a * mn + cc)
    hmx = jnp.where(nonempty, hmx, 0.0)
    hsm = a * sm + cc * cnt.astype(jnp.float32)
    hmean = hsm / jnp.maximum(cnt.astype(jnp.float32), 1.0)
    out = jnp.concatenate([hmx, hmean, hsm], axis=1)
    pool_ref[g:g + 1, :] = out


def _make_pool(hn_mode, Dout):
  Do2 = Dout // 2
  body = functools.partial(_pool_body, hn_mode, Dout)
  if hn_mode == "full":
    hn_shape = jax.ShapeDtypeStruct((N, Dout), jnp.float32)
  elif hn_mode == "split":
    hn_shape = jax.ShapeDtypeStruct((2, N, Do2), jnp.float32)
  else:
    hn_shape = jax.ShapeDtypeStruct((8, 128), jnp.float32)
  return pl.pallas_call(
      body,
      in_specs=[
          pl.BlockSpec(memory_space=pltpu.VMEM),
          pl.BlockSpec(memory_space=pltpu.VMEM),
          pl.BlockSpec(memory_space=pltpu.VMEM),
          pl.BlockSpec(memory_space=pltpu.VMEM),
          pl.BlockSpec(memory_space=pltpu.VMEM),
          pl.BlockSpec(memory_space=pltpu.SMEM),
          pl.BlockSpec(memory_space=pltpu.SMEM),
      ],
      out_specs=[
          pl.BlockSpec(memory_space=pltpu.VMEM),
          pl.BlockSpec(memory_space=pltpu.VMEM),
      ],
      out_shape=[
          jax.ShapeDtypeStruct((G, 3 * Dout), jnp.float32),
          hn_shape,
      ],
  )


def _head_body(x1_ref, x2_ref, x3_ref, Wl1_ref, bl1_ref, Wl2_ref, bl2_ref,
               Wl3_ref, bl3_ref, out_ref):
  z = jnp.concatenate([x1_ref[...], x2_ref[...], x3_ref[...]], axis=1)
  z = jnp.dot(z, Wl1_ref[...], preferred_element_type=jnp.float32) + bl1_ref[...]
  z = jnp.maximum(z, 0.0)
  z = jnp.dot(z, Wl2_ref[...], preferred_element_type=jnp.float32) + bl2_ref[...]
  z = jnp.maximum(z, 0.0)
  z = jnp.dot(z, Wl3_ref[...], preferred_element_type=jnp.float32) + bl3_ref[...]
  out_ref[...] = jax.nn.log_softmax(z, axis=-1)


_spmm12 = _make_spmm(True)
_spmm3 = _make_spmm(False)
_dense1 = _make_dense(False, False, 128, 128)
_dense2 = _make_dense(False, False, 128, 256)
_dense3 = _make_dense(True, True, 256, 384)
_pool1 = _make_pool("full", 128)
_pool2 = _make_pool("split", 256)
_pool3 = _make_pool("none", 384)

KES = (E // NC) // NS // C
KFS = E // NS // C


def kernel(x, edge_index, batch, edge_attr, W1r, W1s, b1, W2r, W2s, b2,
           W3r, W3s, b3, g1, be1, g2, be2, g3, be3, Wl1, bl1, Wl2, bl2,
           Wl3, bl3):
  ew = edge_attr[:, 0]
  pk_e = jnp.transpose(
      edge_index.reshape(2, NC, NS, KES // SB, SB, C), (1, 2, 3, 0, 4, 5))
  pk_f = jnp.transpose(
      edge_index.reshape(2, NS, KFS // SB, SB, C), (1, 2, 0, 3, 4))
  ew16 = jnp.broadcast_to(ew[:, None], (E, 16))
  ew16_e = ew16.reshape(NC, NS, KES, C, 16)
  ew16_f = ew16.reshape(NS, KFS, C, 16)

  starts = jnp.searchsorted(batch, jnp.arange(G, dtype=batch.dtype)
                            ).astype(jnp.int32)
  ends = jnp.searchsorted(batch, jnp.arange(G, dtype=batch.dtype),
                          side="right").astype(jnp.int32)
  counts = ends - starts

  agg1 = _spmm12(x, pk_e, ew16_e)
  t1, s1, q1 = _dense1(agg1, x, W1r, W1s, b1[None, :])
  x1p, hn1 = _pool1(t1, s1, q1, g1[None, :], be1[None, :], starts, counts)

  agg2 = _spmm12(hn1, pk_e, ew16_e)
  t2, s2, q2 = _dense2(agg2, hn1, W2r, W2s, b2[None, :])
  x2p, hn2 = _pool2(t2, s2, q2, g2[None, :], be2[None, :], starts, counts)

  agg3 = _spmm3(hn2, pk_f, ew16_f)
  t3, s3, q3 = _dense3(agg3, hn2, W3r, W3s, b3[None, :])
  x3p, _ = _pool3(t3, s3, q3, g3[None, :], be3[None, :], starts, counts)

  out = pl.pallas_call(
      _head_body,
      out_shape=jax.ShapeDtypeStruct((G, 2), jnp.float32),
  )(x1p, x2p, x3p, Wl1, bl1[None, :], Wl2, bl2[None, :], Wl3, bl3[None, :])
  return out
